# baseline (speedup 1.0000x reference)
import functools

import jax
import jax.numpy as jnp
from jax import lax
from jax.experimental import pallas as pl
from jax.experimental.pallas import tpu as pltpu
from jax.experimental.pallas import tpu_sc as plsc

NUM_CLASSES = 22
EMB_DIM = 512
SLOT_SIZE = 2048
BATCH = 16384

NC, NS = 2, 16
NW = NC * NS
BPW = BATCH // NW
CH = 32
NCH = BPW // CH


def _mlp_body(embs_ref, w1_ref, b1_ref, w2_ref, b2_ref, out_ref):
    h = jnp.dot(embs_ref[...], w1_ref[...], preferred_element_type=jnp.float32)
    h = h + b1_ref[...]
    h = 0.5 * h * (1.0 + lax.erf(h * 0.7071067811865476))
    out = jnp.dot(h, w2_ref[...], preferred_element_type=jnp.float32)
    out_ref[...] = out + b2_ref[...]


def _mlp(embs, W1, b1, W2, b2):
    return pl.pallas_call(
        _mlp_body,
        out_shape=jax.ShapeDtypeStruct((NUM_CLASSES, SLOT_SIZE), jnp.float32),
    )(embs, W1, b1.reshape(1, -1), W2, b2.reshape(1, -1))


_mesh = plsc.VectorSubcoreMesh(core_axis_name="c", subcore_axis_name="s")


@functools.partial(
    pl.kernel,
    mesh=_mesh,
    out_type=jax.ShapeDtypeStruct((BATCH, SLOT_SIZE), jnp.float32),
    scratch_types=[
        pltpu.VMEM((BPW,), jnp.int32),
        pltpu.VMEM((NUM_CLASSES, SLOT_SIZE), jnp.float32),
        pltpu.SemaphoreType.DMA,
    ],
)
def _gather(labels_hbm, table_hbm, out_hbm, idx_v, table_v, sem):
    wid = lax.axis_index("s") * NC + lax.axis_index("c")
    base = wid * BPW
    pltpu.sync_copy(labels_hbm.at[wid], idx_v)
    pltpu.sync_copy(table_hbm, table_v)

    K = 16
    NWIN = BPW // K

    def issue(w):
        labs = idx_v[pl.ds(w * K, K)]
        for j in range(K):
            pltpu.async_copy(table_v.at[labs[j]], out_hbm.at[base + w * K + j], sem)

    def drain(w):
        pltpu.make_async_copy(
            table_v.at[pl.ds(0, K)], out_hbm.at[pl.ds(base + w * K, K)], sem
        ).wait()

    issue(0)

    def outer(w, carry):
        issue(w + 1)
        drain(w)
        return carry

    lax.fori_loop(0, NWIN - 1, outer, 0, unroll=False)
    drain(NWIN - 1)


def kernel(labels, embs, W1, b1, W2, b2):
    proj = _mlp(embs, W1, b1, W2, b2)
    labels2 = labels.astype(jnp.int32).reshape(NW, BPW)
    return _gather(labels2, proj)


# speedup vs baseline: 1.0620x; 1.0620x over previous
import functools

import jax
import jax.numpy as jnp
from jax import lax
from jax.experimental import pallas as pl
from jax.experimental.pallas import tpu as pltpu
from jax.experimental.pallas import tpu_sc as plsc

NUM_CLASSES = 22
EMB_DIM = 512
SLOT_SIZE = 2048
BATCH = 16384

NC, NS = 2, 16
NW = NC * NS
BPW = BATCH // NW
CH = 32
NCH = BPW // CH


def _mlp_body(embs_ref, w1_ref, b1_ref, w2_ref, b2_ref, out_ref):
    h = jnp.dot(embs_ref[...], w1_ref[...], preferred_element_type=jnp.float32)
    h = h + b1_ref[...]
    h = 0.5 * h * (1.0 + lax.erf(h * 0.7071067811865476))
    out = jnp.dot(h, w2_ref[...], preferred_element_type=jnp.float32)
    out_ref[...] = out + b2_ref[...]


def _mlp(embs, W1, b1, W2, b2):
    return pl.pallas_call(
        _mlp_body,
        out_shape=jax.ShapeDtypeStruct((NUM_CLASSES, SLOT_SIZE), jnp.float32),
    )(embs, W1, b1.reshape(1, -1), W2, b2.reshape(1, -1))


_mesh = plsc.VectorSubcoreMesh(core_axis_name="c", subcore_axis_name="s")


@functools.partial(
    pl.kernel,
    mesh=_mesh,
    out_type=jax.ShapeDtypeStruct((BATCH, SLOT_SIZE), jnp.float32),
    scratch_types=[
        pltpu.VMEM((BPW,), jnp.int32),
        pltpu.VMEM((NUM_CLASSES, SLOT_SIZE), jnp.float32),
        pltpu.VMEM_SHARED((NUM_CLASSES, SLOT_SIZE), jnp.float32),
        pltpu.SemaphoreType.DMA,
    ],
)
def _gather(labels_hbm, table_hbm, out_hbm, idx_v, table_v, table_s, sem):
    wid = lax.axis_index("s") * NC + lax.axis_index("c")
    base = wid * BPW
    pltpu.sync_copy(labels_hbm.at[wid], idx_v)

    @pl.when(lax.axis_index("s") == 0)
    def _stage():
        pltpu.sync_copy(table_hbm, table_s)

    plsc.subcore_barrier()
    pltpu.sync_copy(table_s, table_v)

    K = 16

    def outer(w, carry):
        labs = idx_v[pl.ds(w * K, K)]
        for j in range(K):
            pltpu.async_copy(table_v.at[labs[j]], out_hbm.at[base + w * K + j], sem)
        pltpu.make_async_copy(
            table_v.at[pl.ds(0, K)], out_hbm.at[pl.ds(base + w * K, K)], sem
        ).wait()
        return carry

    lax.fori_loop(0, BPW // K, outer, 0, unroll=False)


def kernel(labels, embs, W1, b1, W2, b2):
    proj = _mlp(embs, W1, b1, W2, b2)
    labels2 = labels.astype(jnp.int32).reshape(NW, BPW)
    return _gather(labels2, proj)
